# baseline (speedup 1.0000x reference)
import functools

import jax
import jax.numpy as jnp
from jax import lax
from jax.experimental import pallas as pl
from jax.experimental.pallas import tpu as pltpu
from jax.experimental.pallas import tpu_sc as plsc

N = 10000
D = 128
HF = 128
E = 300000
NP = 10240
EP = 301056
CH = 147


BR = 5120
CQ = 6272
NF = 12


def _hist_body(idx_hbm, zero_hbm, deg_out, buf0, buf1, hist, outv, sh,
               sem0, sem1):
    c = lax.axis_index("c")
    s = lax.axis_index("s")
    a_local = s // 8
    a = c * 2 + a_local
    rng = (s % 8) // 4
    q = s % 4
    base = rng * BR
    lane_off = jax.lax.iota(jnp.int32, 16) * BR
    zero16 = jnp.zeros((16,), jnp.float32)
    one16 = jnp.ones((16,), jnp.float32)

    @pl.when(s < 12)
    def _z():
        pltpu.sync_copy(zero_hbm, sh.at[pl.ds(s * 16, 16)])

    @pl.loop(0, 16 * BR // 16)
    def _zv(v):
        hist[pl.ds(v * 16, 16)] = zero16

    bufs = (buf0, buf1)
    sems = (sem0, sem1)

    def fire(g, b):
        pltpu.async_copy(idx_hbm.at[a, q, pl.ds(g * CQ, CQ)], bufs[b],
                         sems[b])

    def wait(b):
        pltpu.make_async_copy(idx_hbm.at[a, q, pl.ds(0, CQ)], bufs[b],
                              sems[b]).wait()

    fire(0, 0)
    for g in range(NF):
        b = g % 2
        if g < NF - 1:
            fire(g + 1, 1 - b)
        wait(b)
        buf = bufs[b]

        @pl.loop(0, CQ // 128)
        def _row(r):
            for k in range(8):
                idx16 = buf[pl.ds(r * 128 + 16 * k, 16)]
                local = idx16 - base
                m = local.astype(jnp.uint32) < jnp.uint32(BR)
                loc_c = jnp.where(m, local, 0) + lane_off
                val = jnp.where(m, one16, zero16)
                plsc.addupdate_scatter(hist, [loc_c], val)

    @pl.loop(0, 40)
    def _redr(row):
        @pl.loop(0, 8)
        def _redc(g):
            off = row * 128 + g * 16
            acc = hist[pl.ds(off, 16)]
            for l in range(1, 16):
                acc = acc + hist[pl.ds(l * BR + off, 16)]
            outv[row, pl.ds(g * 16, 16)] = acc

    @pl.loop(40, 48)
    def _zd(row):
        for g in range(8):
            outv[row, pl.ds(g * 16, 16)] = zero16

    plsc.subcore_barrier()
    t = a_local * 2 + rng
    for k in range(3):
        rowsel = t * 48 + k * 16 + jax.lax.iota(jnp.int32, 16)
        pltpu.sync_copy(outv.at[pl.ds(k * 16, 16)], sh.at[rowsel], add=True)
    plsc.subcore_barrier()

    @pl.when(s < 12)
    def _wout():
        pltpu.sync_copy(sh.at[pl.ds(s * 16, 16)],
                        deg_out.at[c, pl.ds(s * 16, 16)])


_hist = pl.kernel(
    _hist_body,
    out_type=jax.ShapeDtypeStruct((2, 192, 128), jnp.float32),
    mesh=plsc.VectorSubcoreMesh(core_axis_name="c", subcore_axis_name="s"),
    compiler_params=pltpu.CompilerParams(needs_layout_passes=False),
    scratch_types=[
        pltpu.VMEM((CQ,), jnp.int32),
        pltpu.VMEM((CQ,), jnp.int32),
        pltpu.VMEM((16 * BR,), jnp.float32),
        pltpu.VMEM((48, 128), jnp.float32),
        pltpu.VMEM_SHARED((192, 128), jnp.float32),
        pltpu.SemaphoreType.DMA,
        pltpu.SemaphoreType.DMA,
    ],
)


TEC_E = 18816
ER = 64
EC = TEC_E // ER
NPA = 10016


def _edge_body(m_hbm, sd_hbm, zero_hbm, out_hbm,
               sdb, rb, acc,
               i0, i1, i2, i3, i4, i5, i6, i7,
               g0, g1, g2, g3, s0, s1, s2, s3):
    c = lax.axis_index("c")
    s = lax.axis_index("s")

    @pl.when(s < 15)
    def _z0():
        pltpu.sync_copy(zero_hbm.at[pl.ds(0, 632)],
                        acc.at[pl.ds(s * 632, 632)])

    @pl.when(s == 15)
    def _z1():
        pltpu.sync_copy(zero_hbm.at[pl.ds(0, 536)],
                        acc.at[pl.ds(15 * 632, 536)])

    plsc.subcore_barrier()

    isems = (i0, i1, i2, i3, i4, i5, i6, i7)
    gsems = (g0, g1, g2, g3)
    ssems = (s0, s1, s2, s3)

    def fetch(j, q):
        pltpu.async_copy(sd_hbm.at[c, s, j], sdb.at[q], isems[q])

    def wait_fetch(q):
        pltpu.make_async_copy(sd_hbm.at[c, s, 0], sdb.at[q],
                              isems[q]).wait()

    def gather(q, b):
        pltpu.async_copy(m_hbm.at[sdb.at[q, 0]], rb.at[b], gsems[b])

    def wait_gather(q, b):
        pltpu.make_async_copy(m_hbm.at[sdb.at[q, 0]], rb.at[b],
                              gsems[b]).wait()

    def scat(q, b):
        pltpu.async_copy(rb.at[b], acc.at[sdb.at[q, 1]], ssems[b], add=True)

    def wait_scat(q, b):
        pltpu.make_async_copy(rb.at[b], acc.at[sdb.at[q, 1]],
                              ssems[b]).wait()

    fetch(0, 0)
    fetch(1, 1)
    fetch(2, 2)
    fetch(3, 3)

    def step(j, q, b):
        @pl.when(j >= 4)
        def _w():
            wait_scat((q + 4) % 8, b)

        @pl.when(j + 4 < EC)
        def _p():
            fetch(j + 4, (q + 4) % 8)

        wait_fetch(q)
        gather(q, b)

        @pl.when(j >= 2)
        def _s():
            wait_gather((q + 6) % 8, (b + 2) % 4)
            scat((q + 6) % 8, (b + 2) % 4)

    @pl.loop(0, EC)
    def _chunk(j):
        m8 = lax.rem(j, 8)
        for q in range(8):
            @pl.when(m8 == q)
            def _s(q=q):
                step(j, q, q % 4)

    for j in (EC - 2, EC - 1):
        wait_gather(j % 8, j % 4)
        scat(j % 8, j % 4)
    for j in (EC - 4, EC - 3, EC - 2, EC - 1):
        wait_scat(j % 8, j % 4)
    plsc.subcore_barrier()

    @pl.when(s < 15)
    def _w0():
        pltpu.sync_copy(acc.at[pl.ds(s * 632, 632)],
                        out_hbm.at[c, pl.ds(s * 632, 632)])

    @pl.when(s == 15)
    def _w1():
        pltpu.sync_copy(acc.at[pl.ds(15 * 632, 536)],
                        out_hbm.at[c, pl.ds(15 * 632, 536)])

    plsc.subcore_barrier()


_edge = pl.kernel(
    _edge_body,
    out_type=jax.ShapeDtypeStruct((2, NP, HF), jnp.float32),
    mesh=plsc.VectorSubcoreMesh(core_axis_name="c", subcore_axis_name="s"),
    compiler_params=pltpu.CompilerParams(needs_layout_passes=False),
    scratch_types=[
        pltpu.VMEM((8, 2, ER), jnp.int32),
        pltpu.VMEM((4, ER, HF), jnp.float32),
        pltpu.VMEM_SHARED((NPA, HF), jnp.float32),
        pltpu.SemaphoreType.DMA,
        pltpu.SemaphoreType.DMA,
        pltpu.SemaphoreType.DMA,
        pltpu.SemaphoreType.DMA,
        pltpu.SemaphoreType.DMA,
        pltpu.SemaphoreType.DMA,
        pltpu.SemaphoreType.DMA,
        pltpu.SemaphoreType.DMA,
        pltpu.SemaphoreType.DMA,
        pltpu.SemaphoreType.DMA,
        pltpu.SemaphoreType.DMA,
        pltpu.SemaphoreType.DMA,
        pltpu.SemaphoreType.DMA,
        pltpu.SemaphoreType.DMA,
        pltpu.SemaphoreType.DMA,
        pltpu.SemaphoreType.DMA,
    ],
)


BM = 1024


def _mm_body(x_ref, w_ref, deg_ref, o_ref):
    isd = lax.rsqrt(jnp.maximum(deg_ref[0], 1.0))
    o_ref[0] = jnp.dot(x_ref[0], w_ref[0],
                       preferred_element_type=jnp.float32) * isd


_mm = pl.pallas_call(
    _mm_body,
    grid=(2, NP // BM),
    in_specs=[
        pl.BlockSpec((1, BM, D), lambda t, i: (t, i, 0)),
        pl.BlockSpec((1, D, HF), lambda t, i: (t, 0, 0)),
        pl.BlockSpec((1, BM, 1), lambda t, i: (t, i, 0)),
    ],
    out_specs=pl.BlockSpec((1, BM, HF), lambda t, i: (t, i, 0)),
    out_shape=jax.ShapeDtypeStruct((2, NP, HF), jnp.float32),
)


def _fin_body(p_ref, deg_ref, b_ref, o_ref):
    isd = lax.rsqrt(jnp.maximum(deg_ref[0], 1.0))
    o_ref[0] = p_ref[0] * isd + b_ref[0]


_fin = pl.pallas_call(
    _fin_body,
    grid=(2, NP // BM),
    in_specs=[
        pl.BlockSpec((1, BM, HF), lambda t, i: (t, i, 0)),
        pl.BlockSpec((1, BM, 1), lambda t, i: (t, i, 0)),
        pl.BlockSpec((1, 1, HF), lambda t, i: (t, 0, 0)),
    ],
    out_specs=pl.BlockSpec((1, BM, HF), lambda t, i: (t, i, 0)),
    out_shape=jax.ShapeDtypeStruct((2, NP, HF), jnp.float32),
)


def kernel(x_user, x_item, W_clicks, b_clicks, W_rev, b_rev,
           edge_index_clicks, edge_index_rev):
    pad = jnp.full((EP - E,), N, jnp.int32)
    src_c = jnp.concatenate([edge_index_clicks[0].astype(jnp.int32), pad])
    dst_c = jnp.concatenate([edge_index_clicks[1].astype(jnp.int32), pad])
    src_r = jnp.concatenate([edge_index_rev[0].astype(jnp.int32), pad])
    dst_r = jnp.concatenate([edge_index_rev[1].astype(jnp.int32), pad])

    idx_hist = jnp.stack([src_c, dst_c, src_r, dst_r]).reshape(4, 4, EP // 4)
    zero_deg = jnp.zeros((16, 128), jnp.float32)
    deg = _hist(idx_hist, zero_deg)
    deg = deg.reshape(2, 2, 2, 48, 128)[:, :, :, :40].reshape(4, NP)
    deg_src = jnp.stack([deg[0], deg[2]])[:, :, None]
    deg_dst = jnp.stack([deg[1], deg[3]])[:, :, None]

    xpad = ((0, NP - N), (0, 0))
    X = jnp.stack([jnp.pad(x_user, xpad), jnp.pad(x_item, xpad)])
    Wt = jnp.stack([W_clicks, W_rev])
    B = jnp.stack([b_clicks, b_rev])[:, None, :]

    M = _mm(X, Wt, deg_src)
    gsrc = jnp.stack([src_c, src_r + NP]).reshape(2, 16, EC, 1, ER)
    gdst = jnp.stack([dst_c, dst_r]).reshape(2, 16, EC, 1, ER)
    gsd = jnp.concatenate([gsrc, gdst], axis=3)
    zero_rows = jnp.zeros((NP // 16, HF), jnp.float32)
    P = _edge(M.reshape(2 * NP, HF), gsd, zero_rows)
    O = _fin(P, deg_dst, B)
    return (O[1, :N], O[0, :N])


# speedup vs baseline: 1.0372x; 1.0372x over previous
import functools

import jax
import jax.numpy as jnp
from jax import lax
from jax.experimental import pallas as pl
from jax.experimental.pallas import tpu as pltpu
from jax.experimental.pallas import tpu_sc as plsc

N = 10000
D = 128
HF = 128
E = 300000
NP = 10240
EP = 301056
CH = 147


BR = 5120
BRP = BR + 16
CQ = 6272
NF = 12


def _hist_body(idx_hbm, zero_hbm, deg_out, buf0, buf1, hist, outv, sh,
               sem0, sem1):
    c = lax.axis_index("c")
    s = lax.axis_index("s")
    a_local = s // 8
    a = c * 2 + a_local
    rng = (s % 8) // 4
    q = s % 4
    base = rng * BR
    lane_off = jax.lax.iota(jnp.int32, 16) * BRP
    zero16 = jnp.zeros((16,), jnp.float32)
    one16 = jnp.ones((16,), jnp.float32)

    @pl.when(s < 12)
    def _z():
        pltpu.sync_copy(zero_hbm, sh.at[pl.ds(s * 16, 16)])

    @pl.loop(0, 16 * BRP // 16)
    def _zv(v):
        hist[pl.ds(v * 16, 16)] = zero16

    bufs = (buf0, buf1)
    sems = (sem0, sem1)

    def fire(g, b):
        pltpu.async_copy(idx_hbm.at[a, q, pl.ds(g * CQ, CQ)], bufs[b],
                         sems[b])

    def wait(b):
        pltpu.make_async_copy(idx_hbm.at[a, q, pl.ds(0, CQ)], bufs[b],
                              sems[b]).wait()

    fire(0, 0)
    for g in range(NF):
        b = g % 2
        if g < NF - 1:
            fire(g + 1, 1 - b)
        wait(b)
        buf = bufs[b]

        @pl.loop(0, CQ // 128)
        def _row(r):
            for k in range(8):
                idx16 = buf[pl.ds(r * 128 + 16 * k, 16)]
                local = (idx16 - base).astype(jnp.uint32)
                loc = jnp.minimum(local, jnp.uint32(BR)).astype(jnp.int32)
                plsc.addupdate_scatter(hist, [loc + lane_off], one16)

    @pl.loop(0, 40)
    def _redr(row):
        @pl.loop(0, 8)
        def _redc(g):
            off = row * 128 + g * 16
            acc = hist[pl.ds(off, 16)]
            for l in range(1, 16):
                acc = acc + hist[pl.ds(l * BRP + off, 16)]
            outv[row, pl.ds(g * 16, 16)] = acc

    @pl.loop(40, 48)
    def _zd(row):
        for g in range(8):
            outv[row, pl.ds(g * 16, 16)] = zero16

    plsc.subcore_barrier()
    t = a_local * 2 + rng
    for k in range(3):
        rowsel = t * 48 + k * 16 + jax.lax.iota(jnp.int32, 16)
        pltpu.sync_copy(outv.at[pl.ds(k * 16, 16)], sh.at[rowsel], add=True)
    plsc.subcore_barrier()

    @pl.when(s < 12)
    def _wout():
        pltpu.sync_copy(sh.at[pl.ds(s * 16, 16)],
                        deg_out.at[c, pl.ds(s * 16, 16)])


_hist = pl.kernel(
    _hist_body,
    out_type=jax.ShapeDtypeStruct((2, 192, 128), jnp.float32),
    mesh=plsc.VectorSubcoreMesh(core_axis_name="c", subcore_axis_name="s"),
    compiler_params=pltpu.CompilerParams(needs_layout_passes=False),
    scratch_types=[
        pltpu.VMEM((CQ,), jnp.int32),
        pltpu.VMEM((CQ,), jnp.int32),
        pltpu.VMEM((16 * BRP,), jnp.float32),
        pltpu.VMEM((48, 128), jnp.float32),
        pltpu.VMEM_SHARED((192, 128), jnp.float32),
        pltpu.SemaphoreType.DMA,
        pltpu.SemaphoreType.DMA,
    ],
)


TEC_E = 18816
ER = 128
EC = TEC_E // ER
NPA = 10016


def _edge_body(m_hbm, sd_hbm, zero_hbm, out_hbm,
               sdb, rb, acc,
               i0, i1, i2, i3, i4, i5, g0, g1, g2, s0, s1, s2):
    c = lax.axis_index("c")
    s = lax.axis_index("s")

    @pl.when(s < 15)
    def _z0():
        pltpu.sync_copy(zero_hbm.at[pl.ds(0, 632)],
                        acc.at[pl.ds(s * 632, 632)])

    @pl.when(s == 15)
    def _z1():
        pltpu.sync_copy(zero_hbm.at[pl.ds(0, 536)],
                        acc.at[pl.ds(15 * 632, 536)])

    plsc.subcore_barrier()

    isems = (i0, i1, i2, i3, i4, i5)
    gsems = (g0, g1, g2)
    ssems = (s0, s1, s2)

    def fetch(j, q):
        pltpu.async_copy(sd_hbm.at[c, s, j], sdb.at[q], isems[q])

    def wait_fetch(q):
        pltpu.make_async_copy(sd_hbm.at[c, s, 0], sdb.at[q],
                              isems[q]).wait()

    def gather(q, b):
        pltpu.async_copy(m_hbm.at[sdb.at[q, 0]], rb.at[b], gsems[b])

    def wait_gather(q, b):
        pltpu.make_async_copy(m_hbm.at[sdb.at[q, 0]], rb.at[b],
                              gsems[b]).wait()

    def scat(q, b):
        pltpu.async_copy(rb.at[b], acc.at[sdb.at[q, 1]], ssems[b], add=True)

    def wait_scat(q, b):
        pltpu.make_async_copy(rb.at[b], acc.at[sdb.at[q, 1]],
                              ssems[b]).wait()

    fetch(0, 0)
    fetch(1, 1)
    fetch(2, 2)

    def step(j, q, b):
        @pl.when(j >= 3)
        def _w():
            wait_scat((q + 3) % 6, b)

        @pl.when(j + 3 < EC)
        def _p():
            fetch(j + 3, (q + 3) % 6)

        wait_fetch(q)
        gather(q, b)

        @pl.when(j >= 1)
        def _s():
            wait_gather((q + 5) % 6, (b + 2) % 3)
            scat((q + 5) % 6, (b + 2) % 3)

    @pl.loop(0, EC)
    def _chunk(j):
        m6 = lax.rem(j, 6)
        for q in range(6):
            @pl.when(m6 == q)
            def _s(q=q):
                step(j, q, q % 3)

    wait_gather((EC - 1) % 6, (EC - 1) % 3)
    scat((EC - 1) % 6, (EC - 1) % 3)
    for j in (EC - 3, EC - 2, EC - 1):
        wait_scat(j % 6, j % 3)
    plsc.subcore_barrier()

    @pl.when(s < 15)
    def _w0():
        pltpu.sync_copy(acc.at[pl.ds(s * 632, 632)],
                        out_hbm.at[c, pl.ds(s * 632, 632)])

    @pl.when(s == 15)
    def _w1():
        pltpu.sync_copy(acc.at[pl.ds(15 * 632, 536)],
                        out_hbm.at[c, pl.ds(15 * 632, 536)])

    plsc.subcore_barrier()


_edge = pl.kernel(
    _edge_body,
    out_type=jax.ShapeDtypeStruct((2, NP, HF), jnp.float32),
    mesh=plsc.VectorSubcoreMesh(core_axis_name="c", subcore_axis_name="s"),
    compiler_params=pltpu.CompilerParams(needs_layout_passes=False),
    scratch_types=[
        pltpu.VMEM((6, 2, ER), jnp.int32),
        pltpu.VMEM((3, ER, HF), jnp.float32),
        pltpu.VMEM_SHARED((NPA, HF), jnp.float32),
        pltpu.SemaphoreType.DMA,
        pltpu.SemaphoreType.DMA,
        pltpu.SemaphoreType.DMA,
        pltpu.SemaphoreType.DMA,
        pltpu.SemaphoreType.DMA,
        pltpu.SemaphoreType.DMA,
        pltpu.SemaphoreType.DMA,
        pltpu.SemaphoreType.DMA,
        pltpu.SemaphoreType.DMA,
        pltpu.SemaphoreType.DMA,
        pltpu.SemaphoreType.DMA,
        pltpu.SemaphoreType.DMA,
    ],
)


BM = 1024


def _mm_body(x_ref, w_ref, deg_ref, o_ref):
    isd = lax.rsqrt(jnp.maximum(deg_ref[0], 1.0))
    o_ref[0] = jnp.dot(x_ref[0], w_ref[0],
                       preferred_element_type=jnp.float32) * isd


_mm = pl.pallas_call(
    _mm_body,
    grid=(2, NP // BM),
    in_specs=[
        pl.BlockSpec((1, BM, D), lambda t, i: (t, i, 0)),
        pl.BlockSpec((1, D, HF), lambda t, i: (t, 0, 0)),
        pl.BlockSpec((1, BM, 1), lambda t, i: (t, i, 0)),
    ],
    out_specs=pl.BlockSpec((1, BM, HF), lambda t, i: (t, i, 0)),
    out_shape=jax.ShapeDtypeStruct((2, NP, HF), jnp.float32),
)


def _fin_body(p_ref, deg_ref, b_ref, o_ref):
    isd = lax.rsqrt(jnp.maximum(deg_ref[0], 1.0))
    o_ref[0] = p_ref[0] * isd + b_ref[0]


_fin = pl.pallas_call(
    _fin_body,
    grid=(2, NP // BM),
    in_specs=[
        pl.BlockSpec((1, BM, HF), lambda t, i: (t, i, 0)),
        pl.BlockSpec((1, BM, 1), lambda t, i: (t, i, 0)),
        pl.BlockSpec((1, 1, HF), lambda t, i: (t, 0, 0)),
    ],
    out_specs=pl.BlockSpec((1, BM, HF), lambda t, i: (t, i, 0)),
    out_shape=jax.ShapeDtypeStruct((2, NP, HF), jnp.float32),
)


def kernel(x_user, x_item, W_clicks, b_clicks, W_rev, b_rev,
           edge_index_clicks, edge_index_rev):
    pad = jnp.full((EP - E,), N, jnp.int32)
    src_c = jnp.concatenate([edge_index_clicks[0].astype(jnp.int32), pad])
    dst_c = jnp.concatenate([edge_index_clicks[1].astype(jnp.int32), pad])
    src_r = jnp.concatenate([edge_index_rev[0].astype(jnp.int32), pad])
    dst_r = jnp.concatenate([edge_index_rev[1].astype(jnp.int32), pad])

    idx_hist = jnp.stack([src_c, dst_c, src_r, dst_r]).reshape(4, 4, EP // 4)
    zero_deg = jnp.zeros((16, 128), jnp.float32)
    deg = _hist(idx_hist, zero_deg)
    deg = deg.reshape(2, 2, 2, 48, 128)[:, :, :, :40].reshape(4, NP)
    deg_src = jnp.stack([deg[0], deg[2]])[:, :, None]
    deg_dst = jnp.stack([deg[1], deg[3]])[:, :, None]

    xpad = ((0, NP - N), (0, 0))
    X = jnp.stack([jnp.pad(x_user, xpad), jnp.pad(x_item, xpad)])
    Wt = jnp.stack([W_clicks, W_rev])
    B = jnp.stack([b_clicks, b_rev])[:, None, :]

    M = _mm(X, Wt, deg_src)
    gsrc = jnp.stack([src_c, src_r + NP]).reshape(2, 16, EC, 1, ER)
    gdst = jnp.stack([dst_c, dst_r]).reshape(2, 16, EC, 1, ER)
    gsd = jnp.concatenate([gsrc, gdst], axis=3)
    zero_rows = jnp.zeros((NP // 16, HF), jnp.float32)
    P = _edge(M.reshape(2 * NP, HF), gsd, zero_rows)
    O = _fin(P, deg_dst, B)
    return (O[1, :N], O[0, :N])
